# Initial kernel scaffold; baseline (speedup 1.0000x reference)
import jax
import jax.numpy as jnp
from jax.experimental import pallas as pl


def kernel(x, edge_index, W_l1, W_r1, b1, W_l2, W_r2, b2, W_l3, W_r3, b3, W_fc, b_fc):
    raise NotImplementedError("write your pallas kernel here")


# speedup vs baseline: 6.3628x; 6.3628x over previous
import functools

import jax
import jax.numpy as jnp
from jax import lax
from jax.experimental import pallas as pl
from jax.experimental.pallas import tpu as pltpu
from jax.experimental.pallas import tpu_sc as plsc

N = 10000
D = 64
E = 320000
CHUNK = 128
EPAD = 327680
EROWS = EPAD // CHUNK
NWORK = 32
RPT = EROWS // NWORK
HALF = RPT // 2
NACC = 10016
NPT = 624
NTAIL = N - 16 * NPT
DUMMY = N

_MESH = plsc.VectorSubcoreMesh(core_axis_name="c", subcore_axis_name="s")


def _sc_agg_body(with_deg, *refs):
    if with_deg:
        (y, srci, dsti, z64, z16, ones_h,
         part, degp,
         acc, dega, src_v, dst_v, buf_a, buf_b, ones_v, sem_a, sem_b) = refs
    else:
        (y, srci, dsti, z64,
         part,
         acc, src_v, dst_v, buf_a, buf_b, sem_a, sem_b) = refs

    c = lax.axis_index("c")
    s = lax.axis_index("s")
    wid = s * 2 + c

    r0 = s * NPT
    pltpu.sync_copy(z64.at[pl.ds(r0, NPT)], acc.at[pl.ds(r0, NPT)])
    if with_deg:
        pltpu.sync_copy(z16.at[pl.ds(r0, NPT)], dega.at[pl.ds(r0, NPT)])
        pltpu.sync_copy(ones_h, ones_v)

    @pl.when(s == 15)
    def _():
        t0 = 16 * NPT
        pltpu.sync_copy(z64.at[pl.ds(t0, NTAIL)], acc.at[pl.ds(t0, NTAIL)])
        if with_deg:
            pltpu.sync_copy(z16.at[pl.ds(t0, NTAIL)], dega.at[pl.ds(t0, NTAIL)])

    plsc.subcore_barrier()

    e0 = wid * RPT
    pltpu.sync_copy(srci.at[pl.ds(e0, RPT)], src_v)
    pltpu.sync_copy(dsti.at[pl.ds(e0, RPT)], dst_v)

    def fire(j, buf, sem):
        pltpu.async_copy(y.at[src_v.at[j]], buf, sem)

    def wait(j, buf, sem):
        pltpu.make_async_copy(y.at[src_v.at[j]], buf, sem).wait()

    def scat(j, buf):
        pltpu.sync_copy(buf, acc.at[dst_v.at[j]], add=True)
        if with_deg:
            pltpu.sync_copy(ones_v, dega.at[dst_v.at[j]], add=True)

    fire(0, buf_a, sem_a)
    fire(1, buf_b, sem_b)

    def body(g, carry):
        j0 = 2 * g
        wait(j0, buf_a, sem_a)
        scat(j0, buf_a)
        fire(j0 + 2, buf_a, sem_a)
        wait(j0 + 1, buf_b, sem_b)
        scat(j0 + 1, buf_b)
        fire(j0 + 3, buf_b, sem_b)
        return carry

    lax.fori_loop(0, HALF - 1, body, 0)
    wait(RPT - 2, buf_a, sem_a)
    scat(RPT - 2, buf_a)
    wait(RPT - 1, buf_b, sem_b)
    scat(RPT - 1, buf_b)

    plsc.subcore_barrier()

    pltpu.sync_copy(acc.at[pl.ds(r0, NPT)], part.at[c].at[pl.ds(r0, NPT)])
    if with_deg:
        pltpu.sync_copy(dega.at[pl.ds(r0, NPT)], degp.at[c].at[pl.ds(r0, NPT)])

    @pl.when(s == 15)
    def _():
        t0 = 16 * NPT
        pltpu.sync_copy(acc.at[pl.ds(t0, NTAIL)], part.at[c].at[pl.ds(t0, NTAIL)])
        if with_deg:
            pltpu.sync_copy(dega.at[pl.ds(t0, NTAIL)],
                            degp.at[c].at[pl.ds(t0, NTAIL)])


def _make_sc_agg(with_deg):
    out_type = [jax.ShapeDtypeStruct((2, N, D), jnp.float32)]
    scratch = {
        "acc": pltpu.VMEM_SHARED((NACC, D), jnp.float32),
        "src_v": pltpu.VMEM((RPT, CHUNK), jnp.int32),
        "dst_v": pltpu.VMEM((RPT, CHUNK), jnp.int32),
        "buf_a": pltpu.VMEM((CHUNK, D), jnp.float32),
        "buf_b": pltpu.VMEM((CHUNK, D), jnp.float32),
        "sem_a": pltpu.SemaphoreType.DMA,
        "sem_b": pltpu.SemaphoreType.DMA,
    }
    if with_deg:
        out_type.append(jax.ShapeDtypeStruct((2, N, 16), jnp.float32))
        scratch["dega"] = pltpu.VMEM_SHARED((NACC, 16), jnp.float32)
        scratch["ones_v"] = pltpu.VMEM((CHUNK, 16), jnp.float32)

    order = ["acc", "dega", "src_v", "dst_v", "buf_a", "buf_b", "ones_v",
             "sem_a", "sem_b"]
    scratch_types = [scratch[k] for k in order if k in scratch]

    return pl.kernel(
        functools.partial(_sc_agg_body, with_deg),
        out_type=out_type,
        mesh=_MESH,
        scratch_types=scratch_types,
        compiler_params=pltpu.CompilerParams(use_tc_tiling_on_sc=False),
        name="sc_segsum_deg" if with_deg else "sc_segsum",
    )


_sc_agg_deg = _make_sc_agg(True)
_sc_agg = _make_sc_agg(False)


def _segment_partials(y, srci, dsti, with_deg):
    z64 = jnp.zeros((N, D), jnp.float32)
    if with_deg:
        z16 = jnp.zeros((N, 16), jnp.float32)
        ones_h = jnp.ones((CHUNK, 16), jnp.float32)
        part, degp = _sc_agg_deg(y, srci, dsti, z64, z16, ones_h)
        return part, degp
    (part,) = _sc_agg(y, srci, dsti, z64)
    return part


_BR = 1000
_GRID = (N // _BR,)


def _full(shape):
    return pl.BlockSpec(shape, lambda i: (0,) * len(shape))


def _rows(shape):
    if len(shape) == 3:
        return pl.BlockSpec(shape, lambda i: (0, i, 0))
    return pl.BlockSpec(shape, lambda i: (i, 0))


def _inv_deg(dp):
    deg = dp[0] + dp[1]
    return 1.0 / jnp.maximum(deg[:, 0:1], 1.0)


def _tc1_body(x_ref, wl_ref, wr_ref, y_ref, z_ref):
    xb = x_ref[...]
    y_ref[...] = jnp.dot(xb, wl_ref[...], preferred_element_type=jnp.float32)
    z_ref[...] = jnp.dot(xb, wr_ref[...], preferred_element_type=jnp.float32)


_tc1 = pl.pallas_call(
    _tc1_body,
    grid=_GRID,
    in_specs=[_rows((_BR, 128)), _full((128, D)), _full((128, D))],
    out_specs=[_rows((_BR, D)), _rows((_BR, D))],
    out_shape=[jax.ShapeDtypeStruct((N, D), jnp.float32)] * 2,
)


def _tc2_body(p_ref, dp_ref, z_ref, b_ref, h_ref):
    p = p_ref[...]
    inv = _inv_deg(dp_ref[...])
    mean = (p[0] + p[1]) * inv
    h_ref[...] = jnp.maximum(mean + z_ref[...] + b_ref[...], 0.0)


_tc2 = pl.pallas_call(
    _tc2_body,
    grid=_GRID,
    in_specs=[_rows((2, _BR, D)), _rows((2, _BR, 16)), _rows((_BR, D)),
              _full((1, D))],
    out_specs=_rows((_BR, D)),
    out_shape=jax.ShapeDtypeStruct((N, D), jnp.float32),
)


def _tc3_body(p_ref, dp_ref, h1_ref, wl2_ref, wr2_ref, b2_ref, wl3_ref,
              wr3_ref, y3_ref, z3_ref):
    p = p_ref[...]
    inv = _inv_deg(dp_ref[...])
    mean2 = (p[0] + p[1]) * inv
    h2 = jnp.dot(mean2, wl2_ref[...], preferred_element_type=jnp.float32)
    h2 += jnp.dot(h1_ref[...], wr2_ref[...], preferred_element_type=jnp.float32)
    h2 = jnp.maximum(h2 + b2_ref[...], 0.0)
    y3_ref[...] = jnp.dot(h2, wl3_ref[...], preferred_element_type=jnp.float32)
    z3_ref[...] = jnp.dot(h2, wr3_ref[...], preferred_element_type=jnp.float32)


_tc3 = pl.pallas_call(
    _tc3_body,
    grid=_GRID,
    in_specs=[_rows((2, _BR, D)), _rows((2, _BR, 16)), _rows((_BR, D)),
              _full((D, 128)), _full((D, 128)), _full((1, 128)),
              _full((128, D)), _full((128, D))],
    out_specs=[_rows((_BR, D)), _rows((_BR, D))],
    out_shape=[jax.ShapeDtypeStruct((N, D), jnp.float32)] * 2,
)


def _tc4_body(p_ref, dp_ref, z3_ref, b3_ref, wfc_ref, bfc_ref, o_ref):
    p = p_ref[...]
    inv = _inv_deg(dp_ref[...])
    h3 = jnp.maximum((p[0] + p[1]) * inv + z3_ref[...] + b3_ref[...], 0.0)
    o_ref[...] = jnp.dot(h3, wfc_ref[...],
                         preferred_element_type=jnp.float32) + bfc_ref[...]


_tc4 = pl.pallas_call(
    _tc4_body,
    grid=_GRID,
    in_specs=[_rows((2, _BR, D)), _rows((2, _BR, 16)), _rows((_BR, D)),
              _full((1, D)), _full((D, 128)), _full((1, 128))],
    out_specs=_rows((_BR, 128)),
    out_shape=jax.ShapeDtypeStruct((N, 128), jnp.float32),
)


def kernel(x, edge_index, W_l1, W_r1, b1, W_l2, W_r2, b2, W_l3, W_r3, b3,
           W_fc, b_fc):
    src = edge_index[0]
    dst = edge_index[1]
    pad = EPAD - E
    srci = jnp.concatenate([src, jnp.zeros((pad,), jnp.int32)])
    dsti = jnp.concatenate([dst, jnp.full((pad,), DUMMY, jnp.int32)])
    srci = srci.reshape(EROWS, CHUNK)
    dsti = dsti.reshape(EROWS, CHUNK)

    y1, z1 = _tc1(x, W_l1.T, W_r1.T)
    p1, degp = _segment_partials(y1, srci, dsti, True)
    h1 = _tc2(p1, degp, z1, b1.reshape(1, D))

    p2 = _segment_partials(h1, srci, dsti, False)

    y3, z3 = _tc3(p2, degp, h1, W_l2.T, W_r2.T, b2.reshape(1, 128),
                  W_l3.T, W_r3.T)
    p3 = _segment_partials(y3, srci, dsti, False)

    wfc = jnp.zeros((D, 128), jnp.float32).at[:, :2].set(W_fc.T)
    bfc = jnp.zeros((1, 128), jnp.float32).at[:, :2].set(b_fc.reshape(1, 2))
    out = _tc4(p3, degp, z3, b3.reshape(1, D), wfc, bfc)
    return out[:, :2]
